# baseline (speedup 1.0000x reference)
import jax
import jax.numpy as jnp
from jax.experimental import pallas as pl
from jax.experimental.pallas import tpu as pltpu
from jax.experimental.pallas import tpu_sc as plsc

_ROWS = 128
_ROW_LEN = 160000
_PER_CORE = _ROWS // 2


def _sc_permute_copy(src2d, idx):
    mesh = plsc.ScalarSubcoreMesh(axis_name="c", num_cores=2)

    @pl.kernel(
        out_type=jax.ShapeDtypeStruct((_ROWS, _ROW_LEN), jnp.float32),
        mesh=mesh,
        scratch_types=[
            pltpu.SMEM((_ROWS,), jnp.int32),
            pltpu.SemaphoreType.DMA,
        ],
    )
    def k(src_hbm, idx_hbm, out_hbm, idx_smem, sem):
        c = jax.lax.axis_index("c")
        pltpu.async_copy(idx_hbm, idx_smem, sem).wait()
        base = c * _PER_CORE

        @pl.loop(0, _PER_CORE)
        def _(i):
            row = base + i
            pltpu.make_async_copy(
                src_hbm.at[idx_smem[row]], out_hbm.at[row], sem
            ).start()

        @pl.loop(0, _PER_CORE)
        def _(i):
            row = base + i
            pltpu.make_async_copy(
                src_hbm.at[idx_smem[row]], out_hbm.at[row], sem
            ).wait()

    return k(src2d, idx)


def kernel(sources):
    src2d = sources.reshape(_ROWS, _ROW_LEN)
    perm = jnp.argsort(jax.random.uniform(jax.random.key(42), (_PER_CORE,)))
    idx = jnp.concatenate(
        [perm.astype(jnp.int32), jnp.arange(_PER_CORE, _ROWS, dtype=jnp.int32)]
    )
    out = _sc_permute_copy(src2d, idx)
    return out.reshape(2, _PER_CORE, 1, _ROW_LEN)
